# Initial kernel scaffold; baseline (speedup 1.0000x reference)
import jax
import jax.numpy as jnp
from jax.experimental import pallas as pl


def kernel(features, positions, batch, W_enc, b_enc, Wq, bq, Wk, bk, Wv, bv, Wp1, bp1, Wp2, bp2, Wa1, ba1, Wa2, ba2, W_dec, b_dec):
    raise NotImplementedError("write your pallas kernel here")


# speedup vs baseline: 7.8105x; 7.8105x over previous
import functools

import jax
import jax.numpy as jnp
from jax import lax
from jax.experimental import pallas as pl
from jax.experimental.pallas import tpu as pltpu
from jax.experimental.pallas import tpu_sc as plsc

N = 8192
IN_F = 512
COMP = 128
K = 16
PPAD = 16
RB = 256
PB = 256
NK = N * K


def _encode_knn_body(f_ref, posb_ref, posT_ref, bcol_ref, brow_ref,
                     we_ref, be_ref, wq_ref, bq_ref, wk_ref, bk_ref,
                     wv_ref, bv_ref, wp1_ref,
                     q_ref, k_ref, v_ref, pa_ref, idx_ref):
    h = jnp.dot(f_ref[...], we_ref[...], preferred_element_type=jnp.float32)
    h = h + be_ref[...]
    q_ref[...] = jnp.dot(h, wq_ref[...], preferred_element_type=jnp.float32) + bq_ref[...]
    k_ref[...] = jnp.dot(h, wk_ref[...], preferred_element_type=jnp.float32) + bk_ref[...]
    v_ref[...] = jnp.dot(h, wv_ref[...], preferred_element_type=jnp.float32) + bv_ref[...]

    posb = posb_ref[...]
    pa_ref[...] = jnp.dot(posb, wp1_ref[...], preferred_element_type=jnp.float32)
    posT = posT_ref[...]
    dots = jnp.dot(posb, posT, preferred_element_type=jnp.float32)
    sqb = jnp.sum(posb * posb, axis=1, keepdims=True)
    sqr = jnp.sum(posT * posT, axis=0, keepdims=True)
    d2 = sqb + sqr - 2.0 * dots
    same = bcol_ref[...] == brow_ref[...]
    cand = jnp.where(same, d2, jnp.inf)

    iotaf = lax.broadcasted_iota(jnp.int32, (RB, N), 1).astype(jnp.float32)
    cols = []
    for _ in range(K):
        m = jnp.min(cand, axis=1, keepdims=True)
        tie = jnp.where(cand == m, iotaf, jnp.float32(N))
        amin = jnp.min(tie, axis=1, keepdims=True)
        cols.append(amin)
        cand = jnp.where(iotaf == amin, jnp.inf, cand)
    idx_ref[...] = jnp.concatenate(cols, axis=1).astype(jnp.int32)


def _encode_knn(features, pos16, posT, bcol, brow, W_enc, b_enc,
                Wq, bq, Wk, bk, Wv, bv, Wp1p):
    grid = (N // RB,)
    full = lambda shape: pl.BlockSpec(shape, lambda i: (0, 0))
    rowblk = lambda w: pl.BlockSpec((RB, w), lambda i: (i, 0))
    return pl.pallas_call(
        _encode_knn_body,
        grid=grid,
        in_specs=[
            rowblk(IN_F),
            rowblk(PPAD),
            full((PPAD, N)),
            pl.BlockSpec((RB, 1), lambda i: (i, 0)),
            full((1, N)),
            full((IN_F, COMP)), full((1, COMP)),
            full((COMP, COMP)), full((1, COMP)),
            full((COMP, COMP)), full((1, COMP)),
            full((COMP, COMP)), full((1, COMP)),
            full((PPAD, COMP)),
        ],
        out_specs=[
            rowblk(COMP), rowblk(COMP), rowblk(COMP), rowblk(COMP),
            pl.BlockSpec((RB, K), lambda i: (i, 0)),
        ],
        out_shape=[
            jax.ShapeDtypeStruct((N, COMP), jnp.float32),
            jax.ShapeDtypeStruct((N, COMP), jnp.float32),
            jax.ShapeDtypeStruct((N, COMP), jnp.float32),
            jax.ShapeDtypeStruct((N, COMP), jnp.float32),
            jax.ShapeDtypeStruct((N, K), jnp.int32),
        ],
    )(features, pos16, posT, bcol, brow, W_enc, b_enc, Wq, bq, Wk, bk, Wv, bv, Wp1p)


_NW = 32
_BPW = NK // _NW
_CH = 128
_NCH = _BPW // _CH


def _gather3_body(kf_hbm, v_hbm, p_hbm, idx_hbm,
                  kj_hbm, vj_hbm, pj_hbm,
                  idx_v, kr, vr, pr, sem):
    wid = lax.axis_index("s") * 2 + lax.axis_index("c")

    def body(c, carry):
        base = wid * _BPW + c * _CH
        pltpu.sync_copy(idx_hbm.at[pl.ds(base, _CH)], idx_v)
        c1 = pltpu.async_copy(kf_hbm.at[idx_v], kr, sem)
        c2 = pltpu.async_copy(v_hbm.at[idx_v], vr, sem)
        c3 = pltpu.async_copy(p_hbm.at[idx_v], pr, sem)
        c1.wait()
        c2.wait()
        c3.wait()
        pltpu.sync_copy(kr, kj_hbm.at[pl.ds(base, _CH)])
        pltpu.sync_copy(vr, vj_hbm.at[pl.ds(base, _CH)])
        pltpu.sync_copy(pr, pj_hbm.at[pl.ds(base, _CH)])
        return carry

    lax.fori_loop(0, _NCH, body, 0)


def _gather3(kfeat, v, posA, idx_flat):
    mesh = plsc.VectorSubcoreMesh(core_axis_name="c", subcore_axis_name="s")
    f = functools.partial(
        pl.kernel,
        out_type=[
            jax.ShapeDtypeStruct((NK, COMP), jnp.float32),
            jax.ShapeDtypeStruct((NK, COMP), jnp.float32),
            jax.ShapeDtypeStruct((NK, COMP), jnp.float32),
        ],
        mesh=mesh,
        scratch_types=[
            pltpu.VMEM((_CH,), jnp.int32),
            pltpu.VMEM((_CH, COMP), jnp.float32),
            pltpu.VMEM((_CH, COMP), jnp.float32),
            pltpu.VMEM((_CH, COMP), jnp.float32),
            pltpu.SemaphoreType.DMA,
        ],
    )(_gather3_body)
    return f(kfeat, v, posA, idx_flat)


def _attn_body(kj_ref, vj_ref, pja_ref, q_ref, pia_ref, f_ref,
               bp1_ref, wp2_ref, bp2_ref,
               wa1_ref, ba1_ref, wa2_ref, ba2_ref,
               wd_ref, bd_ref, out_ref):
    nkb = PB * K
    piA = pia_ref[...]
    piAb = jnp.broadcast_to(piA[:, None, :], (PB, K, COMP)).reshape(nkb, COMP)
    pe_h = jnp.maximum(piAb - pja_ref[...] + bp1_ref[...], 0.0)
    pe = jnp.dot(pe_h, wp2_ref[...], preferred_element_type=jnp.float32) + bp2_ref[...]

    qb = jnp.broadcast_to(q_ref[...][:, None, :], (PB, K, COMP)).reshape(nkb, COMP)
    a = qb - kj_ref[...] + pe
    a_h = jnp.maximum(jnp.dot(a, wa1_ref[...], preferred_element_type=jnp.float32) + ba1_ref[...], 0.0)
    a = jnp.dot(a_h, wa2_ref[...], preferred_element_type=jnp.float32) + ba2_ref[...]

    a3 = a.reshape(PB, K, COMP)
    mx = jnp.max(a3, axis=1, keepdims=True)
    e = jnp.exp(a3 - mx)
    s = jnp.sum(e, axis=1, keepdims=True)
    w = e / s
    vpe = (vj_ref[...] + pe).reshape(PB, K, COMP)
    agg = jnp.sum(w * vpe, axis=1)

    y = jnp.dot(agg, wd_ref[...], preferred_element_type=jnp.float32) + bd_ref[...]
    out_ref[...] = f_ref[...] + y


def _attention(kj, vj, pjA, q, piA, features,
               bp1, Wp2, bp2, Wa1, ba1, Wa2, ba2, W_dec, b_dec):
    grid = (N // PB,)
    full = lambda shape: pl.BlockSpec(shape, lambda i: (0, 0))
    nkblk = lambda w: pl.BlockSpec((PB * K, w), lambda i: (i, 0))
    pblk = lambda w: pl.BlockSpec((PB, w), lambda i: (i, 0))
    return pl.pallas_call(
        _attn_body,
        grid=grid,
        in_specs=[
            nkblk(COMP), nkblk(COMP), nkblk(COMP),
            pblk(COMP), pblk(COMP), pblk(IN_F),
            full((1, COMP)),
            full((COMP, COMP)), full((1, COMP)),
            full((COMP, COMP)), full((1, COMP)),
            full((COMP, COMP)), full((1, COMP)),
            full((COMP, IN_F)), full((1, IN_F)),
        ],
        out_specs=pblk(IN_F),
        out_shape=jax.ShapeDtypeStruct((N, IN_F), jnp.float32),
    )(kj, vj, pjA, q, piA, features,
      bp1, Wp2, bp2, Wa1, ba1, Wa2, ba2, W_dec, b_dec)


def kernel(features, positions, batch, W_enc, b_enc, Wq, bq, Wk, bk, Wv, bv,
           Wp1, bp1, Wp2, bp2, Wa1, ba1, Wa2, ba2, W_dec, b_dec):
    pos16 = jnp.pad(positions, ((0, 0), (0, PPAD - 3)))
    posT = pos16.T
    batchf = batch.astype(jnp.float32)
    bcol = batchf.reshape(N, 1)
    brow = batchf.reshape(1, N)
    row = lambda b: b.reshape(1, -1)

    Wp1p = jnp.pad(Wp1, ((0, PPAD - 3), (0, 0)))
    q, kfeat, v, posA, idx = _encode_knn(
        features, pos16, posT, bcol, brow, W_enc, row(b_enc),
        Wq, row(bq), Wk, row(bk), Wv, row(bv), Wp1p)

    idx_flat = idx.reshape(NK)
    kj, vj, pjA = _gather3(kfeat, v, posA, idx_flat)

    out = _attention(kj, vj, pjA, q, posA, features,
                     row(bp1), Wp2, row(bp2),
                     Wa1, row(ba1), Wa2, row(ba2), W_dec, row(b_dec))
    return (out, positions, batch)


# speedup vs baseline: 11.0954x; 1.4206x over previous
import functools

import jax
import jax.numpy as jnp
from jax import lax
from jax.experimental import pallas as pl
from jax.experimental.pallas import tpu as pltpu
from jax.experimental.pallas import tpu_sc as plsc

N = 8192
IN_F = 512
COMP = 128
K = 16
PPAD = 16
RB = 256
PB = 256
NK = N * K


CW = 512


def _encode_knn_body(cs_ref, ce_ref,
                     f_ref, posb_ref, posT_ref, bcol_ref, brow_ref,
                     we_ref, be_ref, wq_ref, bq_ref, wk_ref, bk_ref,
                     wv_ref, bv_ref, wp1_ref,
                     q_ref, k_ref, v_ref, pa_ref, idx_ref):
    h = jnp.dot(f_ref[...], we_ref[...], preferred_element_type=jnp.float32)
    h = h + be_ref[...]
    q_ref[...] = jnp.dot(h, wq_ref[...], preferred_element_type=jnp.float32) + bq_ref[...]
    k_ref[...] = jnp.dot(h, wk_ref[...], preferred_element_type=jnp.float32) + bk_ref[...]
    v_ref[...] = jnp.dot(h, wv_ref[...], preferred_element_type=jnp.float32) + bv_ref[...]

    posb = posb_ref[...]
    pa_ref[...] = jnp.dot(posb, wp1_ref[...], preferred_element_type=jnp.float32)
    sqb = jnp.sum(posb * posb, axis=1, keepdims=True)
    bcol = bcol_ref[...]
    i = pl.program_id(0)
    cs = cs_ref[i]
    ce = ce_ref[i]

    init_v = jnp.full((RB, K), jnp.inf, jnp.float32)
    init_i = jnp.broadcast_to(
        (-1.0 - lax.broadcasted_iota(jnp.int32, (1, K), 1).astype(jnp.float32)),
        (RB, K))
    iota_c = lax.broadcasted_iota(jnp.int32, (RB, CW), 1)

    def chunk_body(c, carry):
        vals, idxs = carry
        pcols = posT_ref[:, pl.ds(c * CW, CW)]
        dots = jnp.dot(posb, pcols, preferred_element_type=jnp.float32)
        sqr = jnp.sum(pcols * pcols, axis=0, keepdims=True)
        d2 = sqb + sqr - 2.0 * dots
        same = bcol == brow_ref[:, pl.ds(c * CW, CW)]
        cand = jnp.where(same, d2, jnp.inf)
        cidx = (c * CW + iota_c).astype(jnp.float32)
        wv = jnp.concatenate([vals, cand], axis=1)
        wi = jnp.concatenate([idxs, cidx], axis=1)
        nv, ni = [], []
        for _ in range(K):
            m = jnp.min(wv, axis=1, keepdims=True)
            tie = jnp.where(wv == m, wi, jnp.float32(N))
            amin = jnp.min(tie, axis=1, keepdims=True)
            nv.append(m)
            ni.append(amin)
            wv = jnp.where(tie == amin, jnp.inf, wv)
        return jnp.concatenate(nv, axis=1), jnp.concatenate(ni, axis=1)

    vals, idxs = lax.fori_loop(cs, ce, chunk_body, (init_v, init_i))
    idx_ref[...] = jnp.clip(idxs, 0.0, jnp.float32(N - 1)).astype(jnp.int32)


def _encode_knn(cs, ce, features, pos16, posT, bcol, brow, W_enc, b_enc,
                Wq, bq, Wk, bk, Wv, bv, Wp1p):
    grid = (N // RB,)
    full = lambda shape: pl.BlockSpec(shape, lambda i, s0, s1: (0, 0))
    rowblk = lambda w: pl.BlockSpec((RB, w), lambda i, s0, s1: (i, 0))
    return pl.pallas_call(
        _encode_knn_body,
        grid_spec=pltpu.PrefetchScalarGridSpec(
            num_scalar_prefetch=2,
            grid=grid,
            in_specs=[
                rowblk(IN_F),
                rowblk(PPAD),
                full((PPAD, N)),
                pl.BlockSpec((RB, 1), lambda i, s0, s1: (i, 0)),
                full((1, N)),
                full((IN_F, COMP)), full((1, COMP)),
                full((COMP, COMP)), full((1, COMP)),
                full((COMP, COMP)), full((1, COMP)),
                full((COMP, COMP)), full((1, COMP)),
                full((PPAD, COMP)),
            ],
            out_specs=[
                rowblk(COMP), rowblk(COMP), rowblk(COMP), rowblk(COMP),
                pl.BlockSpec((RB, K), lambda i, s0, s1: (i, 0)),
            ],
        ),
        out_shape=[
            jax.ShapeDtypeStruct((N, COMP), jnp.float32),
            jax.ShapeDtypeStruct((N, COMP), jnp.float32),
            jax.ShapeDtypeStruct((N, COMP), jnp.float32),
            jax.ShapeDtypeStruct((N, COMP), jnp.float32),
            jax.ShapeDtypeStruct((N, K), jnp.int32),
        ],
    )(cs, ce, features, pos16, posT, bcol, brow, W_enc, b_enc,
      Wq, bq, Wk, bk, Wv, bv, Wp1p)


_NW = 32
_BPW = NK // _NW
_CH = 128
_NCH = _BPW // _CH


def _gather3_body(kf_hbm, v_hbm, p_hbm, idx_hbm,
                  kj_hbm, vj_hbm, pj_hbm,
                  idx_v, kr, vr, pr, sem):
    wid = lax.axis_index("s") * 2 + lax.axis_index("c")

    def body(c, carry):
        base = wid * _BPW + c * _CH
        pltpu.sync_copy(idx_hbm.at[pl.ds(base, _CH)], idx_v)
        c1 = pltpu.async_copy(kf_hbm.at[idx_v], kr, sem)
        c2 = pltpu.async_copy(v_hbm.at[idx_v], vr, sem)
        c3 = pltpu.async_copy(p_hbm.at[idx_v], pr, sem)
        c1.wait()
        c2.wait()
        c3.wait()
        pltpu.sync_copy(kr, kj_hbm.at[pl.ds(base, _CH)])
        pltpu.sync_copy(vr, vj_hbm.at[pl.ds(base, _CH)])
        pltpu.sync_copy(pr, pj_hbm.at[pl.ds(base, _CH)])
        return carry

    lax.fori_loop(0, _NCH, body, 0)


def _gather3(kfeat, v, posA, idx_flat):
    mesh = plsc.VectorSubcoreMesh(core_axis_name="c", subcore_axis_name="s")
    f = functools.partial(
        pl.kernel,
        out_type=[
            jax.ShapeDtypeStruct((NK, COMP), jnp.float32),
            jax.ShapeDtypeStruct((NK, COMP), jnp.float32),
            jax.ShapeDtypeStruct((NK, COMP), jnp.float32),
        ],
        mesh=mesh,
        scratch_types=[
            pltpu.VMEM((_CH,), jnp.int32),
            pltpu.VMEM((_CH, COMP), jnp.float32),
            pltpu.VMEM((_CH, COMP), jnp.float32),
            pltpu.VMEM((_CH, COMP), jnp.float32),
            pltpu.SemaphoreType.DMA,
        ],
    )(_gather3_body)
    return f(kfeat, v, posA, idx_flat)


def _attn_body(kj_ref, vj_ref, pja_ref, q_ref, pia_ref, f_ref,
               bp1_ref, wp2_ref, bp2_ref,
               wa1_ref, ba1_ref, wa2_ref, ba2_ref,
               wd_ref, bd_ref, out_ref):
    nkb = PB * K
    piA = pia_ref[...]
    piAb = jnp.broadcast_to(piA[:, None, :], (PB, K, COMP)).reshape(nkb, COMP)
    pe_h = jnp.maximum(piAb - pja_ref[...] + bp1_ref[...], 0.0)
    pe = jnp.dot(pe_h, wp2_ref[...], preferred_element_type=jnp.float32) + bp2_ref[...]

    qb = jnp.broadcast_to(q_ref[...][:, None, :], (PB, K, COMP)).reshape(nkb, COMP)
    a = qb - kj_ref[...] + pe
    a_h = jnp.maximum(jnp.dot(a, wa1_ref[...], preferred_element_type=jnp.float32) + ba1_ref[...], 0.0)
    a = jnp.dot(a_h, wa2_ref[...], preferred_element_type=jnp.float32) + ba2_ref[...]

    a3 = a.reshape(PB, K, COMP)
    mx = jnp.max(a3, axis=1, keepdims=True)
    e = jnp.exp(a3 - mx)
    s = jnp.sum(e, axis=1, keepdims=True)
    w = e / s
    vpe = (vj_ref[...] + pe).reshape(PB, K, COMP)
    agg = jnp.sum(w * vpe, axis=1)

    y = jnp.dot(agg, wd_ref[...], preferred_element_type=jnp.float32) + bd_ref[...]
    out_ref[...] = f_ref[...] + y


def _attention(kj, vj, pjA, q, piA, features,
               bp1, Wp2, bp2, Wa1, ba1, Wa2, ba2, W_dec, b_dec):
    grid = (N // PB,)
    full = lambda shape: pl.BlockSpec(shape, lambda i: (0, 0))
    nkblk = lambda w: pl.BlockSpec((PB * K, w), lambda i: (i, 0))
    pblk = lambda w: pl.BlockSpec((PB, w), lambda i: (i, 0))
    return pl.pallas_call(
        _attn_body,
        grid=grid,
        in_specs=[
            nkblk(COMP), nkblk(COMP), nkblk(COMP),
            pblk(COMP), pblk(COMP), pblk(IN_F),
            full((1, COMP)),
            full((COMP, COMP)), full((1, COMP)),
            full((COMP, COMP)), full((1, COMP)),
            full((COMP, COMP)), full((1, COMP)),
            full((COMP, IN_F)), full((1, IN_F)),
        ],
        out_specs=pblk(IN_F),
        out_shape=jax.ShapeDtypeStruct((N, IN_F), jnp.float32),
    )(kj, vj, pjA, q, piA, features,
      bp1, Wp2, bp2, Wa1, ba1, Wa2, ba2, W_dec, b_dec)


def kernel(features, positions, batch, W_enc, b_enc, Wq, bq, Wk, bk, Wv, bv,
           Wp1, bp1, Wp2, bp2, Wa1, ba1, Wa2, ba2, W_dec, b_dec):
    pos16 = jnp.pad(positions, ((0, 0), (0, PPAD - 3)))
    posT = pos16.T
    batchf = batch.astype(jnp.float32)
    bcol = batchf.reshape(N, 1)
    brow = batchf.reshape(1, N)
    row = lambda b: b.reshape(1, -1)

    Wp1p = jnp.pad(Wp1, ((0, PPAD - 3), (0, 0)))
    blk = jnp.arange(N // RB)
    b_lo = batch[blk * RB]
    b_hi = batch[blk * RB + RB - 1]
    col_start = jnp.searchsorted(batch, b_lo, side="left")
    col_end = jnp.searchsorted(batch, b_hi, side="right")
    cs = (col_start // CW).astype(jnp.int32)
    ce = ((col_end + CW - 1) // CW).astype(jnp.int32)
    q, kfeat, v, posA, idx = _encode_knn(
        cs, ce, features, pos16, posT, bcol, brow, W_enc, row(b_enc),
        Wq, row(bq), Wk, row(bk), Wv, row(bv), Wp1p)

    idx_flat = idx.reshape(NK)
    kj, vj, pjA = _gather3(kfeat, v, posA, idx_flat)

    out = _attention(kj, vj, pjA, q, posA, features,
                     row(bp1), Wp2, row(bp2),
                     Wa1, row(ba1), Wa2, row(ba2), W_dec, row(b_dec))
    return (out, positions, batch)


# speedup vs baseline: 11.5307x; 1.0392x over previous
import functools

import jax
import jax.numpy as jnp
from jax import lax
from jax.experimental import pallas as pl
from jax.experimental.pallas import tpu as pltpu
from jax.experimental.pallas import tpu_sc as plsc

N = 8192
IN_F = 512
COMP = 128
K = 16
PPAD = 16
RB = 256
PB = 256
NK = N * K


CW = 512


def _encode_knn_body(cs_ref, ce_ref,
                     f_ref, posb_ref, posT_ref, bcol_ref, brow_ref,
                     we_ref, be_ref, wq_ref, bq_ref, wk_ref, bk_ref,
                     wv_ref, bv_ref, wp1_ref,
                     q_ref, kvp_ref, idx_ref):
    h = jnp.dot(f_ref[...], we_ref[...], preferred_element_type=jnp.float32)
    h = h + be_ref[...]
    q_ref[...] = jnp.dot(h, wq_ref[...], preferred_element_type=jnp.float32) + bq_ref[...]
    kvp_ref[:, 0:COMP] = jnp.dot(h, wk_ref[...], preferred_element_type=jnp.float32) + bk_ref[...]
    kvp_ref[:, COMP:2 * COMP] = jnp.dot(h, wv_ref[...], preferred_element_type=jnp.float32) + bv_ref[...]

    posb = posb_ref[...]
    kvp_ref[:, 2 * COMP:3 * COMP] = jnp.dot(posb, wp1_ref[...], preferred_element_type=jnp.float32)
    sqb = jnp.sum(posb * posb, axis=1, keepdims=True)
    bcol = bcol_ref[...]
    i = pl.program_id(0)
    cs = cs_ref[i]
    ce = ce_ref[i]

    init_v = jnp.full((RB, K), jnp.inf, jnp.float32)
    init_i = jnp.broadcast_to(
        (-1.0 - lax.broadcasted_iota(jnp.int32, (1, K), 1).astype(jnp.float32)),
        (RB, K))
    iota_c = lax.broadcasted_iota(jnp.int32, (RB, CW), 1)

    def chunk_body(c, carry):
        vals, idxs = carry
        pcols = posT_ref[:, pl.ds(c * CW, CW)]
        dots = jnp.dot(posb, pcols, preferred_element_type=jnp.float32)
        sqr = jnp.sum(pcols * pcols, axis=0, keepdims=True)
        d2 = sqb + sqr - 2.0 * dots
        same = bcol == brow_ref[:, pl.ds(c * CW, CW)]
        cand = jnp.where(same, d2, jnp.inf)
        cidx = (c * CW + iota_c).astype(jnp.float32)
        wv = jnp.concatenate([vals, cand], axis=1)
        wi = jnp.concatenate([idxs, cidx], axis=1)
        nv, ni = [], []
        for _ in range(K):
            m = jnp.min(wv, axis=1, keepdims=True)
            tie = jnp.where(wv == m, wi, jnp.float32(N))
            amin = jnp.min(tie, axis=1, keepdims=True)
            nv.append(m)
            ni.append(amin)
            wv = jnp.where(tie == amin, jnp.inf, wv)
        return jnp.concatenate(nv, axis=1), jnp.concatenate(ni, axis=1)

    vals, idxs = lax.fori_loop(cs, ce, chunk_body, (init_v, init_i))
    idx_ref[...] = jnp.clip(idxs, 0.0, jnp.float32(N - 1)).astype(jnp.int32)


def _encode_knn(cs, ce, features, pos16, posT, bcol, brow, W_enc, b_enc,
                Wq, bq, Wk, bk, Wv, bv, Wp1p):
    grid = (N // RB,)
    full = lambda shape: pl.BlockSpec(shape, lambda i, s0, s1: (0, 0))
    rowblk = lambda w: pl.BlockSpec((RB, w), lambda i, s0, s1: (i, 0))
    return pl.pallas_call(
        _encode_knn_body,
        grid_spec=pltpu.PrefetchScalarGridSpec(
            num_scalar_prefetch=2,
            grid=grid,
            in_specs=[
                rowblk(IN_F),
                rowblk(PPAD),
                full((PPAD, N)),
                pl.BlockSpec((RB, 1), lambda i, s0, s1: (i, 0)),
                full((1, N)),
                full((IN_F, COMP)), full((1, COMP)),
                full((COMP, COMP)), full((1, COMP)),
                full((COMP, COMP)), full((1, COMP)),
                full((COMP, COMP)), full((1, COMP)),
                full((PPAD, COMP)),
            ],
            out_specs=[
                rowblk(COMP), rowblk(3 * COMP),
                pl.BlockSpec((RB, K), lambda i, s0, s1: (i, 0)),
            ],
        ),
        out_shape=[
            jax.ShapeDtypeStruct((N, COMP), jnp.float32),
            jax.ShapeDtypeStruct((N, 3 * COMP), jnp.float32),
            jax.ShapeDtypeStruct((N, K), jnp.int32),
        ],
    )(cs, ce, features, pos16, posT, bcol, brow, W_enc, b_enc,
      Wq, bq, Wk, bk, Wv, bv, Wp1p)


_NW = 32
_BPW = NK // _NW
_CH = 128
_NCH = _BPW // _CH
_TW = 3 * COMP


def _gather_body(kvp_hbm, idx_hbm, out_hbm,
                 idx_v, bufa, bufb, gsa, gsb, ssa, ssb):
    wid = lax.axis_index("s") * 2 + lax.axis_index("c")
    base = wid * _BPW
    pltpu.sync_copy(idx_hbm.at[pl.ds(base, _BPW)], idx_v)

    def pair(p, carry):
        ca = 2 * p
        cb = 2 * p + 1
        ga = pltpu.async_copy(
            kvp_hbm.at[idx_v.at[pl.ds(ca * _CH, _CH)]], bufa, gsa)
        gb = pltpu.async_copy(
            kvp_hbm.at[idx_v.at[pl.ds(cb * _CH, _CH)]], bufb, gsb)
        ga.wait()
        sa = pltpu.async_copy(bufa, out_hbm.at[pl.ds(base + ca * _CH, _CH)], ssa)
        gb.wait()
        sb = pltpu.async_copy(bufb, out_hbm.at[pl.ds(base + cb * _CH, _CH)], ssb)
        sa.wait()
        sb.wait()
        return carry

    lax.fori_loop(0, _NCH // 2, pair, 0)


def _gather(kvp, idx_flat):
    mesh = plsc.VectorSubcoreMesh(core_axis_name="c", subcore_axis_name="s")
    f = functools.partial(
        pl.kernel,
        out_type=jax.ShapeDtypeStruct((NK, _TW), jnp.float32),
        mesh=mesh,
        scratch_types=[
            pltpu.VMEM((_BPW,), jnp.int32),
            pltpu.VMEM((_CH, _TW), jnp.float32),
            pltpu.VMEM((_CH, _TW), jnp.float32),
            pltpu.SemaphoreType.DMA,
            pltpu.SemaphoreType.DMA,
            pltpu.SemaphoreType.DMA,
            pltpu.SemaphoreType.DMA,
        ],
    )(_gather_body)
    return f(kvp, idx_flat)


def _attn_body(kj_ref, vj_ref, pja_ref, q_ref, pia_ref, f_ref,
               bp1_ref, wp2_ref, bp2_ref,
               wa1_ref, ba1_ref, wa2_ref, ba2_ref,
               wd_ref, bd_ref, out_ref):
    nkb = PB * K
    piA = pia_ref[...]
    piAb = jnp.broadcast_to(piA[:, None, :], (PB, K, COMP)).reshape(nkb, COMP)
    pe_h = jnp.maximum(piAb - pja_ref[...] + bp1_ref[...], 0.0)
    pe = jnp.dot(pe_h, wp2_ref[...], preferred_element_type=jnp.float32) + bp2_ref[...]

    qb = jnp.broadcast_to(q_ref[...][:, None, :], (PB, K, COMP)).reshape(nkb, COMP)
    a = qb - kj_ref[...] + pe
    a_h = jnp.maximum(jnp.dot(a, wa1_ref[...], preferred_element_type=jnp.float32) + ba1_ref[...], 0.0)
    a = jnp.dot(a_h, wa2_ref[...], preferred_element_type=jnp.float32) + ba2_ref[...]

    a3 = a.reshape(PB, K, COMP)
    mx = jnp.max(a3, axis=1, keepdims=True)
    e = jnp.exp(a3 - mx)
    s = jnp.sum(e, axis=1, keepdims=True)
    w = e / s
    vpe = (vj_ref[...] + pe).reshape(PB, K, COMP)
    agg = jnp.sum(w * vpe, axis=1)

    y = jnp.dot(agg, wd_ref[...], preferred_element_type=jnp.float32) + bd_ref[...]
    out_ref[...] = f_ref[...] + y


def _attention(kvpj, q, kvp, features,
               bp1, Wp2, bp2, Wa1, ba1, Wa2, ba2, W_dec, b_dec):
    grid = (N // PB,)
    full = lambda shape: pl.BlockSpec(shape, lambda i: (0, 0))
    nkcol = lambda c: pl.BlockSpec((PB * K, COMP), lambda i, c=c: (i, c))
    pblk = lambda w: pl.BlockSpec((PB, w), lambda i: (i, 0))
    return pl.pallas_call(
        _attn_body,
        grid=grid,
        in_specs=[
            nkcol(0), nkcol(1), nkcol(2),
            pblk(COMP),
            pl.BlockSpec((PB, COMP), lambda i: (i, 2)),
            pblk(IN_F),
            full((1, COMP)),
            full((COMP, COMP)), full((1, COMP)),
            full((COMP, COMP)), full((1, COMP)),
            full((COMP, COMP)), full((1, COMP)),
            full((COMP, IN_F)), full((1, IN_F)),
        ],
        out_specs=pblk(IN_F),
        out_shape=jax.ShapeDtypeStruct((N, IN_F), jnp.float32),
    )(kvpj, kvpj, kvpj, q, kvp, features,
      bp1, Wp2, bp2, Wa1, ba1, Wa2, ba2, W_dec, b_dec)


def kernel(features, positions, batch, W_enc, b_enc, Wq, bq, Wk, bk, Wv, bv,
           Wp1, bp1, Wp2, bp2, Wa1, ba1, Wa2, ba2, W_dec, b_dec):
    pos16 = jnp.pad(positions, ((0, 0), (0, PPAD - 3)))
    posT = pos16.T
    batchf = batch.astype(jnp.float32)
    bcol = batchf.reshape(N, 1)
    brow = batchf.reshape(1, N)
    row = lambda b: b.reshape(1, -1)

    Wp1p = jnp.pad(Wp1, ((0, PPAD - 3), (0, 0)))
    blk = jnp.arange(N // RB)
    b_lo = batch[blk * RB]
    b_hi = batch[blk * RB + RB - 1]
    col_start = jnp.searchsorted(batch, b_lo, side="left")
    col_end = jnp.searchsorted(batch, b_hi, side="right")
    cs = (col_start // CW).astype(jnp.int32)
    ce = ((col_end + CW - 1) // CW).astype(jnp.int32)
    q, kvp, idx = _encode_knn(
        cs, ce, features, pos16, posT, bcol, brow, W_enc, row(b_enc),
        Wq, row(bq), Wk, row(bk), Wv, row(bv), Wp1p)

    idx_flat = idx.reshape(NK)
    kvpj = _gather(kvp, idx_flat)

    out = _attention(kvpj, q, kvp, features,
                     row(bp1), Wp2, row(bp2),
                     Wa1, row(ba1), Wa2, row(ba2), W_dec, row(b_dec))
    return (out, positions, batch)
